# baseline (speedup 1.0000x reference)
import functools

import jax
import jax.numpy as jnp
from jax import lax
from jax.experimental import pallas as pl
from jax.experimental.pallas import tpu as pltpu
from jax.experimental.pallas import tpu_sc as plsc

NC = 2
NS = 16
NW = NC * NS
B_EDGE = 128
NBUF = 4
ZTILES = 10
DEG_COLS = 8
PAD_ROWS = 8

_mesh = plsc.VectorSubcoreMesh(core_axis_name="c", subcore_axis_name="s")
_sc_params = pltpu.CompilerParams(use_tc_tiling_on_sc=False)


def _make_deg_kernel(n, k_steps):
    zrows = n // ZTILES

    @functools.partial(
        pl.kernel,
        out_type=jax.ShapeDtypeStruct((NC, n, DEG_COLS), jnp.float32),
        mesh=_mesh,
        scratch_types=[
            pltpu.VMEM((k_steps, B_EDGE), jnp.int32),
            pltpu.VMEM((B_EDGE, DEG_COLS), jnp.float32),
            pltpu.VMEM_SHARED((n + PAD_ROWS, DEG_COLS), jnp.float32),
            pltpu.SemaphoreType.DMA,
        ],
        compiler_params=_sc_params,
    )
    def deg_kernel(dst_hbm, ones_hbm, zeros_hbm, out_hbm, dst_v, ones_v, acc_sh, sem):
        cid = lax.axis_index("c")
        sid = lax.axis_index("s")
        wid = sid * NC + cid
        pltpu.sync_copy(dst_hbm.at[wid], dst_v)
        pltpu.sync_copy(ones_hbm, ones_v)

        @pl.when(sid < ZTILES)
        def _():
            pltpu.sync_copy(zeros_hbm.at[pl.ds(sid * zrows, zrows)],
                            acc_sh.at[pl.ds(sid * zrows, zrows)])

        plsc.subcore_barrier()

        def body(j, carry):
            pltpu.async_copy(ones_v, acc_sh.at[dst_v.at[j]], sem, add=True)
            return carry

        lax.fori_loop(0, k_steps, body, 0)

        def drain(j, carry):
            pltpu.make_async_copy(ones_v, acc_sh.at[dst_v.at[0]], sem).wait()
            return carry

        lax.fori_loop(0, k_steps, drain, 0)
        plsc.subcore_barrier()

        @pl.when(sid < ZTILES)
        def _():
            pltpu.sync_copy(acc_sh.at[pl.ds(sid * zrows, zrows)],
                            out_hbm.at[cid, pl.ds(sid * zrows, zrows)])

    return deg_kernel


def _make_edge_scatter_kernel(n, d, k_steps):
    zrows = n // ZTILES

    @functools.partial(
        pl.kernel,
        out_type=jax.ShapeDtypeStruct((NC, n, d), jnp.float32),
        mesh=_mesh,
        scratch_types=(
            [pltpu.VMEM((k_steps, B_EDGE), jnp.int32),
             pltpu.VMEM((k_steps, B_EDGE), jnp.int32)]
            + [pltpu.VMEM((B_EDGE, d), jnp.float32) for _ in range(NBUF)]
            + [pltpu.VMEM_SHARED((n + PAD_ROWS, d), jnp.float32)]
            + [pltpu.SemaphoreType.DMA for _ in range(2 * NBUF)]
        ),
        compiler_params=_sc_params,
    )
    def edge_kernel(hs_hbm, src_hbm, dst_hbm, zeros_hbm, out_hbm,
                    src_v, dst_v, *rest):
        bufs = rest[:NBUF]
        acc_sh = rest[NBUF]
        gsem = rest[NBUF + 1:NBUF + 1 + NBUF]
        ssem = rest[NBUF + 1 + NBUF:]
        cid = lax.axis_index("c")
        sid = lax.axis_index("s")
        wid = sid * NC + cid
        pltpu.sync_copy(src_hbm.at[wid], src_v)
        pltpu.sync_copy(dst_hbm.at[wid], dst_v)

        @pl.when(sid < ZTILES)
        def _():
            pltpu.sync_copy(zeros_hbm.at[pl.ds(sid * zrows, zrows)],
                            acc_sh.at[pl.ds(sid * zrows, zrows)])

        plsc.subcore_barrier()

        for b in range(NBUF):
            pltpu.async_copy(hs_hbm.at[src_v.at[b]], bufs[b], gsem[b])

        def body(i, carry):
            j0 = i * NBUF
            for b in range(NBUF):
                pltpu.make_async_copy(hs_hbm.at[src_v.at[0]], bufs[b], gsem[b]).wait()
                pltpu.async_copy(bufs[b], acc_sh.at[dst_v.at[j0 + b]], ssem[b], add=True)
            for b in range(NBUF):
                pltpu.make_async_copy(bufs[b], acc_sh.at[dst_v.at[0]], ssem[b]).wait()
                pltpu.async_copy(hs_hbm.at[src_v.at[j0 + NBUF + b]], bufs[b], gsem[b])
            return carry

        lax.fori_loop(0, k_steps // NBUF - 1, body, 0)

        j0 = k_steps - NBUF
        descs = []
        for b in range(NBUF):
            pltpu.make_async_copy(hs_hbm.at[src_v.at[0]], bufs[b], gsem[b]).wait()
            descs.append(
                pltpu.async_copy(bufs[b], acc_sh.at[dst_v.at[j0 + b]], ssem[b], add=True))
        for dsc in descs:
            dsc.wait()
        plsc.subcore_barrier()

        @pl.when(sid < ZTILES)
        def _():
            pltpu.sync_copy(acc_sh.at[pl.ds(sid * zrows, zrows)],
                            out_hbm.at[cid, pl.ds(sid * zrows, zrows)])

    return edge_kernel


def _tc1_body(degp_ref, x_ref, w1_ref, hs_ref, dis_ref):
    dp = degp_ref[...]
    deg = dp[0, :, 0:1] + dp[1, :, 0:1] + 1.0
    dis = lax.rsqrt(deg)
    h = jnp.dot(x_ref[...], w1_ref[...], preferred_element_type=jnp.float32)
    hs_ref[...] = h * dis
    dis_ref[...] = dis


def _tc2_body(accp_ref, hs1_ref, dis_ref, w2_ref, b1_ref, hs2_ref):
    a = accp_ref[...]
    dis = dis_ref[...]
    z = jnp.maximum((a[0] + a[1] + hs1_ref[...]) * dis + b1_ref[...], 0.0)
    h2 = jnp.dot(z, w2_ref[...], preferred_element_type=jnp.float32)
    hs2_ref[...] = h2 * dis


def _tc3_body(accp_ref, hs2_ref, dis_ref, b2_ref, out_ref):
    a = accp_ref[...]
    out_ref[...] = (a[0] + a[1] + hs2_ref[...]) * dis_ref[...] + b2_ref[...]


def kernel(x, edge_index, W1, b1, W2, b2):
    n, d_in = x.shape
    d_hid = W1.shape[1]
    d_out = W2.shape[1]
    e = edge_index.shape[1]
    assert n % ZTILES == 0
    blocks = -(-e // (NW * B_EDGE))
    k_steps = -(-blocks // NBUF) * NBUF
    e_pad = NW * B_EDGE * k_steps
    pad = e_pad - e

    src = jnp.concatenate(
        [edge_index[0], jnp.zeros((pad,), jnp.int32)]).reshape(NW, k_steps, B_EDGE)
    dst = jnp.concatenate(
        [edge_index[1], jnp.full((pad,), n, jnp.int32)]).reshape(NW, k_steps, B_EDGE)
    ones8 = jnp.ones((B_EDGE, DEG_COLS), jnp.float32)
    zeros8 = jnp.zeros((n, DEG_COLS), jnp.float32)
    zeros_h = jnp.zeros((n, d_hid), jnp.float32)
    zeros_o = jnp.zeros((n, d_out), jnp.float32)

    degp = _make_deg_kernel(n, k_steps)(dst, ones8, zeros8)

    hs1, dis = pl.pallas_call(
        _tc1_body,
        out_shape=(jax.ShapeDtypeStruct((n, d_hid), jnp.float32),
                   jax.ShapeDtypeStruct((n, 1), jnp.float32)),
    )(degp, x, W1)

    acc1 = _make_edge_scatter_kernel(n, d_hid, k_steps)(hs1, src, dst, zeros_h)

    hs2 = pl.pallas_call(
        _tc2_body,
        out_shape=jax.ShapeDtypeStruct((n, d_out), jnp.float32),
    )(acc1, hs1, dis, W2, b1.reshape(1, d_hid))

    acc2 = _make_edge_scatter_kernel(n, d_out, k_steps)(hs2, src, dst, zeros_o)

    out = pl.pallas_call(
        _tc3_body,
        out_shape=jax.ShapeDtypeStruct((n, d_out), jnp.float32),
    )(acc2, hs2, dis, b2.reshape(1, d_out))

    return out


# speedup vs baseline: 2.2452x; 2.2452x over previous
import functools

import jax
import jax.numpy as jnp
from jax import lax
from jax.experimental import pallas as pl
from jax.experimental.pallas import tpu as pltpu
from jax.experimental.pallas import tpu_sc as plsc

NC = 2
NS = 16
NW = NC * NS
B_EDGE = 80
NBUF = 5
ZTILES = 10
DEG_COLS = 8
PAD_ROWS = 8

_mesh = plsc.VectorSubcoreMesh(core_axis_name="c", subcore_axis_name="s")
_sc_params = pltpu.CompilerParams(use_tc_tiling_on_sc=False)


def _make_deg_kernel(n, k_steps):
    zrows = n // ZTILES

    @functools.partial(
        pl.kernel,
        out_type=jax.ShapeDtypeStruct((NC, n, DEG_COLS), jnp.float32),
        mesh=_mesh,
        scratch_types=[
            pltpu.VMEM((k_steps, B_EDGE), jnp.int32),
            pltpu.VMEM((B_EDGE, DEG_COLS), jnp.float32),
            pltpu.VMEM_SHARED((n + PAD_ROWS, DEG_COLS), jnp.float32),
            pltpu.SemaphoreType.DMA,
        ],
        compiler_params=_sc_params,
    )
    def deg_kernel(dst_hbm, ones_hbm, zeros_hbm, out_hbm, dst_v, ones_v, acc_sh, sem):
        cid = lax.axis_index("c")
        sid = lax.axis_index("s")
        wid = sid * NC + cid
        pltpu.sync_copy(dst_hbm.at[wid], dst_v)
        pltpu.sync_copy(ones_hbm, ones_v)

        @pl.when(sid < ZTILES)
        def _():
            pltpu.sync_copy(zeros_hbm.at[pl.ds(sid * zrows, zrows)],
                            acc_sh.at[pl.ds(sid * zrows, zrows)])

        plsc.subcore_barrier()

        def body(j, carry):
            pltpu.async_copy(ones_v, acc_sh.at[dst_v.at[j]], sem, add=True)
            return carry

        lax.fori_loop(0, k_steps, body, 0)

        def drain(j, carry):
            pltpu.make_async_copy(ones_v, acc_sh.at[dst_v.at[0]], sem).wait()
            return carry

        lax.fori_loop(0, k_steps, drain, 0)
        plsc.subcore_barrier()

        @pl.when(sid < ZTILES)
        def _():
            pltpu.sync_copy(acc_sh.at[pl.ds(sid * zrows, zrows)],
                            out_hbm.at[cid, pl.ds(sid * zrows, zrows)])

    return deg_kernel


def _make_edge_scatter_kernel(n, d, k_steps):
    zrows = n // ZTILES

    @functools.partial(
        pl.kernel,
        out_type=jax.ShapeDtypeStruct((NC, n, d), jnp.float32),
        mesh=_mesh,
        scratch_types=(
            [pltpu.VMEM((k_steps, B_EDGE), jnp.int32),
             pltpu.VMEM((k_steps, B_EDGE), jnp.int32)]
            + [pltpu.VMEM((B_EDGE, d), jnp.float32) for _ in range(NBUF)]
            + [pltpu.VMEM_SHARED((n + PAD_ROWS, d), jnp.float32)]
            + [pltpu.SemaphoreType.DMA for _ in range(2 * NBUF)]
        ),
        compiler_params=_sc_params,
    )
    def edge_kernel(hs_hbm, src_hbm, dst_hbm, zeros_hbm, out_hbm,
                    src_v, dst_v, *rest):
        bufs = rest[:NBUF]
        acc_sh = rest[NBUF]
        gsem = rest[NBUF + 1:NBUF + 1 + NBUF]
        ssem = rest[NBUF + 1 + NBUF:]
        cid = lax.axis_index("c")
        sid = lax.axis_index("s")
        wid = sid * NC + cid
        pltpu.sync_copy(src_hbm.at[wid], src_v)
        pltpu.sync_copy(dst_hbm.at[wid], dst_v)

        @pl.when(sid < ZTILES)
        def _():
            pltpu.sync_copy(zeros_hbm.at[pl.ds(sid * zrows, zrows)],
                            acc_sh.at[pl.ds(sid * zrows, zrows)])

        plsc.subcore_barrier()

        for b in range(NBUF):
            pltpu.async_copy(hs_hbm.at[src_v.at[b]], bufs[b], gsem[b])

        def body(i, carry):
            j0 = i * NBUF
            for b in range(NBUF):
                pltpu.make_async_copy(hs_hbm.at[src_v.at[0]], bufs[b], gsem[b]).wait()
                pltpu.async_copy(bufs[b], acc_sh.at[dst_v.at[j0 + b]], ssem[b], add=True)
            for b in range(NBUF):
                pltpu.make_async_copy(bufs[b], acc_sh.at[dst_v.at[0]], ssem[b]).wait()
                pltpu.async_copy(hs_hbm.at[src_v.at[j0 + NBUF + b]], bufs[b], gsem[b])
            return carry

        lax.fori_loop(0, k_steps // NBUF - 1, body, 0)

        j0 = k_steps - NBUF
        descs = []
        for b in range(NBUF):
            pltpu.make_async_copy(hs_hbm.at[src_v.at[0]], bufs[b], gsem[b]).wait()
            descs.append(
                pltpu.async_copy(bufs[b], acc_sh.at[dst_v.at[j0 + b]], ssem[b], add=True))
        for dsc in descs:
            dsc.wait()
        plsc.subcore_barrier()

        @pl.when(sid < ZTILES)
        def _():
            pltpu.sync_copy(acc_sh.at[pl.ds(sid * zrows, zrows)],
                            out_hbm.at[cid, pl.ds(sid * zrows, zrows)])

    return edge_kernel


def _tc1_body(degp_ref, x_ref, w1_ref, hs_ref, dis_ref):
    dp = degp_ref[...]
    deg = dp[0, :, 0:1] + dp[1, :, 0:1] + 1.0
    dis = lax.rsqrt(deg)
    h = jnp.dot(x_ref[...], w1_ref[...], preferred_element_type=jnp.float32)
    hs_ref[...] = h * dis
    dis_ref[...] = dis


def _tc2_body(accp_ref, hs1_ref, dis_ref, w2_ref, b1_ref, hs2_ref):
    a = accp_ref[...]
    dis = dis_ref[...]
    z = jnp.maximum((a[0] + a[1] + hs1_ref[...]) * dis + b1_ref[...], 0.0)
    h2 = jnp.dot(z, w2_ref[...], preferred_element_type=jnp.float32)
    hs2_ref[...] = h2 * dis


def _tc3_body(accp_ref, hs2_ref, dis_ref, b2_ref, out_ref):
    a = accp_ref[...]
    out_ref[...] = (a[0] + a[1] + hs2_ref[...]) * dis_ref[...] + b2_ref[...]


def kernel(x, edge_index, W1, b1, W2, b2):
    n, d_in = x.shape
    d_hid = W1.shape[1]
    d_out = W2.shape[1]
    e = edge_index.shape[1]
    assert n % ZTILES == 0
    blocks = -(-e // (NW * B_EDGE))
    k_steps = -(-blocks // NBUF) * NBUF
    e_pad = NW * B_EDGE * k_steps
    pad = e_pad - e

    src = jnp.concatenate(
        [edge_index[0], jnp.zeros((pad,), jnp.int32)]).reshape(NW, k_steps, B_EDGE)
    dst = jnp.concatenate(
        [edge_index[1], jnp.full((pad,), n, jnp.int32)]).reshape(NW, k_steps, B_EDGE)
    ones8 = jnp.ones((B_EDGE, DEG_COLS), jnp.float32)
    zeros8 = jnp.zeros((n, DEG_COLS), jnp.float32)
    zeros_h = jnp.zeros((n, d_hid), jnp.float32)
    zeros_o = jnp.zeros((n, d_out), jnp.float32)

    degp = _make_deg_kernel(n, k_steps)(dst, ones8, zeros8)

    hs1, dis = pl.pallas_call(
        _tc1_body,
        out_shape=(jax.ShapeDtypeStruct((n, d_hid), jnp.float32),
                   jax.ShapeDtypeStruct((n, 1), jnp.float32)),
    )(degp, x, W1)

    acc1 = _make_edge_scatter_kernel(n, d_hid, k_steps)(hs1, src, dst, zeros_h)

    hs2 = pl.pallas_call(
        _tc2_body,
        out_shape=jax.ShapeDtypeStruct((n, d_out), jnp.float32),
    )(acc1, hs1, dis, W2, b1.reshape(1, d_hid))

    acc2 = _make_edge_scatter_kernel(n, d_out, k_steps)(hs2, src, dst, zeros_o)

    out = pl.pallas_call(
        _tc3_body,
        out_shape=jax.ShapeDtypeStruct((n, d_out), jnp.float32),
    )(acc2, hs2, dis, b2.reshape(1, d_out))

    return out


# speedup vs baseline: 2.3693x; 1.0552x over previous
import functools

import jax
import jax.numpy as jnp
from jax import lax
from jax.experimental import pallas as pl
from jax.experimental.pallas import tpu as pltpu
from jax.experimental.pallas import tpu_sc as plsc

NC = 2
NS = 16
NW = NC * NS
B_EDGE = 80
NBUF = 5
ZTILES = 10
DEG_COLS = 8

_mesh = plsc.VectorSubcoreMesh(core_axis_name="c", subcore_axis_name="s")
_sc_params = pltpu.CompilerParams(use_tc_tiling_on_sc=False)


def _make_deg_kernel(n, k_steps):
    zrows = n // ZTILES

    @functools.partial(
        pl.kernel,
        out_type=jax.ShapeDtypeStruct((NC, n, DEG_COLS), jnp.float32),
        mesh=_mesh,
        scratch_types=[
            pltpu.VMEM((k_steps, B_EDGE), jnp.int32),
            pltpu.VMEM((B_EDGE, DEG_COLS), jnp.float32),
            pltpu.VMEM_SHARED((n, DEG_COLS), jnp.float32),
            pltpu.SemaphoreType.DMA,
        ],
        compiler_params=_sc_params,
    )
    def deg_kernel(ei_hbm, ones_hbm, zeros_hbm, out_hbm, dst_v, ones_v, acc_sh, sem):
        cid = lax.axis_index("c")
        sid = lax.axis_index("s")
        wid = sid * NC + cid
        pltpu.sync_copy(ei_hbm.at[1, wid], dst_v)
        pltpu.sync_copy(ones_hbm, ones_v)

        @pl.when(sid < ZTILES)
        def _():
            pltpu.sync_copy(zeros_hbm, acc_sh.at[pl.ds(sid * zrows, zrows)])

        plsc.subcore_barrier()

        def body(j, carry):
            pltpu.async_copy(ones_v, acc_sh.at[dst_v.at[j]], sem, add=True)
            return carry

        lax.fori_loop(0, k_steps, body, 0)

        def drain(j, carry):
            pltpu.make_async_copy(ones_v, acc_sh.at[dst_v.at[0]], sem).wait()
            return carry

        lax.fori_loop(0, k_steps, drain, 0)
        plsc.subcore_barrier()

        @pl.when(sid < ZTILES)
        def _():
            pltpu.sync_copy(acc_sh.at[pl.ds(sid * zrows, zrows)],
                            out_hbm.at[cid, pl.ds(sid * zrows, zrows)])

    return deg_kernel


def _make_edge_scatter_kernel(n, d, k_steps):
    zrows = n // ZTILES

    @functools.partial(
        pl.kernel,
        out_type=jax.ShapeDtypeStruct((NC, n, d), jnp.float32),
        mesh=_mesh,
        scratch_types=(
            [pltpu.VMEM((k_steps, B_EDGE), jnp.int32),
             pltpu.VMEM((k_steps, B_EDGE), jnp.int32)]
            + [pltpu.VMEM((B_EDGE, d), jnp.float32) for _ in range(NBUF)]
            + [pltpu.VMEM_SHARED((n, d), jnp.float32)]
            + [pltpu.SemaphoreType.DMA for _ in range(2 * NBUF)]
        ),
        compiler_params=_sc_params,
    )
    def edge_kernel(hs_hbm, ei_hbm, zeros_hbm, out_hbm, src_v, dst_v, *rest):
        bufs = rest[:NBUF]
        acc_sh = rest[NBUF]
        gsem = rest[NBUF + 1:NBUF + 1 + NBUF]
        ssem = rest[NBUF + 1 + NBUF:]
        cid = lax.axis_index("c")
        sid = lax.axis_index("s")
        wid = sid * NC + cid
        pltpu.sync_copy(ei_hbm.at[0, wid], src_v)
        pltpu.sync_copy(ei_hbm.at[1, wid], dst_v)

        @pl.when(sid < ZTILES)
        def _():
            pltpu.sync_copy(zeros_hbm, acc_sh.at[pl.ds(sid * zrows, zrows)])

        plsc.subcore_barrier()

        for b in range(NBUF):
            pltpu.async_copy(hs_hbm.at[src_v.at[b]], bufs[b], gsem[b])

        def body(i, carry):
            j0 = i * NBUF
            for b in range(NBUF):
                pltpu.make_async_copy(hs_hbm.at[src_v.at[0]], bufs[b], gsem[b]).wait()
                pltpu.async_copy(bufs[b], acc_sh.at[dst_v.at[j0 + b]], ssem[b], add=True)
            for b in range(NBUF):
                pltpu.make_async_copy(bufs[b], acc_sh.at[dst_v.at[0]], ssem[b]).wait()
                pltpu.async_copy(hs_hbm.at[src_v.at[j0 + NBUF + b]], bufs[b], gsem[b])
            return carry

        lax.fori_loop(0, k_steps // NBUF - 1, body, 0)

        j0 = k_steps - NBUF
        descs = []
        for b in range(NBUF):
            pltpu.make_async_copy(hs_hbm.at[src_v.at[0]], bufs[b], gsem[b]).wait()
            descs.append(
                pltpu.async_copy(bufs[b], acc_sh.at[dst_v.at[j0 + b]], ssem[b], add=True))
        for dsc in descs:
            dsc.wait()
        plsc.subcore_barrier()

        @pl.when(sid < ZTILES)
        def _():
            pltpu.sync_copy(acc_sh.at[pl.ds(sid * zrows, zrows)],
                            out_hbm.at[cid, pl.ds(sid * zrows, zrows)])

    return edge_kernel


def _tc1_body(degp_ref, x_ref, w1_ref, hs_ref, dis_ref):
    dp = degp_ref[...]
    deg = dp[0, :, 0:1] + dp[1, :, 0:1] + 1.0
    dis = lax.rsqrt(deg)
    h = jnp.dot(x_ref[...], w1_ref[...], preferred_element_type=jnp.float32)
    hs_ref[...] = h * dis
    dis_ref[...] = dis


def _tc2_body(accp_ref, hs1_ref, dis_ref, w2_ref, b1_ref, hs2_ref):
    a = accp_ref[...]
    dis = dis_ref[...]
    z = jnp.maximum((a[0] + a[1] + hs1_ref[...]) * dis + b1_ref[...], 0.0)
    h2 = jnp.dot(z, w2_ref[...], preferred_element_type=jnp.float32)
    hs2_ref[...] = h2 * dis


def _tc3_body(accp_ref, hs2_ref, dis_ref, b2_ref, out_ref):
    a = accp_ref[...]
    out_ref[...] = (a[0] + a[1] + hs2_ref[...]) * dis_ref[...] + b2_ref[...]


def kernel(x, edge_index, W1, b1, W2, b2):
    n, d_in = x.shape
    d_hid = W1.shape[1]
    d_out = W2.shape[1]
    e = edge_index.shape[1]
    assert n % ZTILES == 0
    assert e % (NW * B_EDGE * NBUF) == 0
    k_steps = e // (NW * B_EDGE)
    zrows = n // ZTILES

    ei = edge_index.reshape(2, NW, k_steps, B_EDGE)
    ones8 = jnp.ones((B_EDGE, DEG_COLS), jnp.float32)
    zeros8 = jnp.zeros((zrows, DEG_COLS), jnp.float32)
    zeros_h = jnp.zeros((zrows, d_hid), jnp.float32)
    zeros_o = jnp.zeros((zrows, d_out), jnp.float32)

    degp = _make_deg_kernel(n, k_steps)(ei, ones8, zeros8)

    hs1, dis = pl.pallas_call(
        _tc1_body,
        out_shape=(jax.ShapeDtypeStruct((n, d_hid), jnp.float32),
                   jax.ShapeDtypeStruct((n, 1), jnp.float32)),
    )(degp, x, W1)

    acc1 = _make_edge_scatter_kernel(n, d_hid, k_steps)(hs1, ei, zeros_h)

    hs2 = pl.pallas_call(
        _tc2_body,
        out_shape=jax.ShapeDtypeStruct((n, d_out), jnp.float32),
    )(acc1, hs1, dis, W2, b1.reshape(1, d_hid))

    acc2 = _make_edge_scatter_kernel(n, d_out, k_steps)(hs2, ei, zeros_o)

    out = pl.pallas_call(
        _tc3_body,
        out_shape=jax.ShapeDtypeStruct((n, d_out), jnp.float32),
    )(acc2, hs2, dis, b2.reshape(1, d_out))

    return out
